# Initial kernel scaffold; baseline (speedup 1.0000x reference)
import jax
import jax.numpy as jnp
from jax.experimental import pallas as pl


def kernel(x, edge_index, W1, b1, W2, b2, gamma, beta):
    raise NotImplementedError("write your pallas kernel here")


# speedup vs baseline: 3.2867x; 3.2867x over previous
import functools

import jax
import jax.numpy as jnp
from jax import lax
from jax.experimental import pallas as pl
from jax.experimental.pallas import tpu as pltpu
from jax.experimental.pallas import tpu_sc as plsc

_N = 10000
_E = 320000
_D = 128

_NC = 2
_NS = 16
_NW = _NC * _NS
_CHUNK = 128
_CPT = 80
_E_PAD = _NW * _CPT * _CHUNK
_NROW = 640
_N_PAD = _NS * _NROW

_LANES = 16

_sc_mesh = plsc.VectorSubcoreMesh(core_axis_name="c", subcore_axis_name="s")


def _make_sc_agg(with_deg):
  out_type = [jax.ShapeDtypeStruct((_NC * _N_PAD, _D), jnp.float32)]
  if with_deg:
    out_type.append(jax.ShapeDtypeStruct((_NC * _N_PAD,), jnp.float32))

  scratch = [
      pltpu.VMEM((_CPT, _CHUNK), jnp.int32),
      pltpu.VMEM((_CPT, _CHUNK), jnp.int32),
      pltpu.VMEM((_CHUNK, _D), jnp.float32),
      pltpu.VMEM((_CHUNK,), jnp.float32),
      pltpu.VMEM((_CHUNK,), jnp.float32),
      pltpu.VMEM_SHARED((_N_PAD, _D), jnp.float32),
      pltpu.VMEM_SHARED((_N_PAD,), jnp.float32),
      pltpu.SemaphoreType.DMA,
  ]

  def body(x_hbm, srcr_hbm, dstr_hbm, *rest):
    if with_deg:
      agg_hbm, deg_hbm = rest[0], rest[1]
      rest = rest[2:]
    else:
      agg_hbm = rest[0]
      rest = rest[1:]
    idx_s, idx_d, rows, vec1d, z1d, agg_sh, deg_sh, sem = rest

    cid = lax.axis_index("c")
    sid = lax.axis_index("s")
    wid = cid * _NS + sid
    zero16 = jnp.zeros((_LANES,), jnp.float32)
    one16 = jnp.ones((_LANES,), jnp.float32)

    def zrow(r, _):
      for c in range(_D // _LANES):
        rows[r, pl.ds(c * _LANES, _LANES)] = zero16
      return _
    lax.fori_loop(0, _CHUNK, zrow, 0)

    if with_deg:
      def fill1d(r, _):
        vec1d[pl.ds(r * _LANES, _LANES)] = one16
        z1d[pl.ds(r * _LANES, _LANES)] = zero16
        return _
      lax.fori_loop(0, _CHUNK // _LANES, fill1d, 0)

    row0 = sid * _NROW
    for k in range(_NROW // _CHUNK):
      pltpu.sync_copy(rows, agg_sh.at[pl.ds(row0 + k * _CHUNK, _CHUNK)])
      if with_deg:
        pltpu.sync_copy(z1d, deg_sh.at[pl.ds(row0 + k * _CHUNK, _CHUNK)])

    pltpu.sync_copy(srcr_hbm.at[pl.ds(wid * _CPT, _CPT)], idx_s)
    pltpu.sync_copy(dstr_hbm.at[pl.ds(wid * _CPT, _CPT)], idx_d)

    plsc.subcore_barrier()

    def step(j, _):
      pltpu.async_copy(x_hbm.at[idx_s.at[j]], rows, sem).wait()
      pltpu.sync_copy(rows, agg_sh.at[idx_d.at[j]], add=True)
      if with_deg:
        pltpu.sync_copy(vec1d, deg_sh.at[idx_d.at[j]], add=True)
      return _
    lax.fori_loop(0, _CPT, step, 0)

    plsc.subcore_barrier()

    off = cid * _N_PAD + row0
    pltpu.sync_copy(agg_sh.at[pl.ds(row0, _NROW)], agg_hbm.at[pl.ds(off, _NROW)])
    if with_deg:
      pltpu.sync_copy(deg_sh.at[pl.ds(row0, _NROW)],
                      deg_hbm.at[pl.ds(off, _NROW)])

  return pl.kernel(body, out_type=out_type, mesh=_sc_mesh,
                   scratch_types=scratch)


_sc_agg_deg = _make_sc_agg(True)
_sc_agg = _make_sc_agg(False)

_BLK = 512
_GRID = _N_PAD // _BLK


def _dot(a, w):
  return lax.dot_general(a, w, (((1,), (0,)), ((), ())),
                         precision=lax.Precision.HIGHEST,
                         preferred_element_type=jnp.float32)


def _tc_mid_body(ap, dp, w, bb, o):
  a = ap[0] + ap[1]
  dg = jnp.clip(dp[0] + dp[1], 1.0, None)
  s = a / dg
  o[...] = jnp.tanh(_dot(s, w[...]) + bb[...])


def _tc_final_body(ap, dp, w, bb, xb, gb, betab, o):
  a = ap[0] + ap[1]
  dg = jnp.clip(dp[0] + dp[1], 1.0, None)
  s = a / dg
  h = jnp.tanh(_dot(s, w[...]) + bb[...])
  r = xb[...] + h
  m = jnp.mean(r, axis=1, keepdims=True)
  c = r - m
  v = jnp.mean(c * c, axis=1, keepdims=True)
  o[...] = c * lax.rsqrt(v + 1e-5) * gb[...] + betab[...]


_spec_agg = pl.BlockSpec((2, _BLK, _D), lambda i: (0, i, 0))
_spec_deg = pl.BlockSpec((2, _BLK, 1), lambda i: (0, i, 0))
_spec_w = pl.BlockSpec((_D, _D), lambda i: (0, 0))
_spec_row = pl.BlockSpec((1, _D), lambda i: (0, 0))
_spec_x = pl.BlockSpec((_BLK, _D), lambda i: (i, 0))

_tc_mid = pl.pallas_call(
    _tc_mid_body,
    grid=(_GRID,),
    in_specs=[_spec_agg, _spec_deg, _spec_w, _spec_row],
    out_specs=_spec_x,
    out_shape=jax.ShapeDtypeStruct((_N_PAD, _D), jnp.float32),
)

_tc_final = pl.pallas_call(
    _tc_final_body,
    grid=(_GRID,),
    in_specs=[_spec_agg, _spec_deg, _spec_w, _spec_row, _spec_x, _spec_row,
              _spec_row],
    out_specs=_spec_x,
    out_shape=jax.ShapeDtypeStruct((_N_PAD, _D), jnp.float32),
)


def kernel(x, edge_index, W1, b1, W2, b2, gamma, beta):
  src = edge_index[0]
  dst = edge_index[1]
  pad = _E_PAD - _E
  src_p = jnp.concatenate(
      [src, jnp.zeros((pad,), jnp.int32)]).reshape(_E_PAD // _CHUNK, _CHUNK)
  dst_p = jnp.concatenate(
      [dst, jnp.full((pad,), _N, jnp.int32)]).reshape(_E_PAD // _CHUNK, _CHUNK)

  agg1, deg = _sc_agg_deg(x, src_p, dst_p)
  agg1 = agg1.reshape(_NC, _N_PAD, _D)
  deg = deg.reshape(_NC, _N_PAD, 1)

  h1 = _tc_mid(agg1, deg, W1, b1.reshape(1, _D))

  agg2, = _sc_agg(h1, src_p, dst_p)
  agg2 = agg2.reshape(_NC, _N_PAD, _D)

  x_pad = jnp.concatenate([x, jnp.zeros((_N_PAD - _N, _D), jnp.float32)])
  out = _tc_final(agg2, deg, W2, b2.reshape(1, _D), x_pad,
                  gamma.reshape(1, _D), beta.reshape(1, _D))
  return out[:_N]


# speedup vs baseline: 3.5991x; 1.0951x over previous
import functools

import jax
import jax.numpy as jnp
from jax import lax
from jax.experimental import pallas as pl
from jax.experimental.pallas import tpu as pltpu
from jax.experimental.pallas import tpu_sc as plsc

_N = 10000
_E = 320000
_D = 128

_NC = 2
_NS = 16
_NW = _NC * _NS
_CHUNK = 128
_CPT = 80
_E_PAD = _NW * _CPT * _CHUNK
_NROW = 640
_N_PAD = _NS * _NROW

_LANES = 16

_sc_mesh = plsc.VectorSubcoreMesh(core_axis_name="c", subcore_axis_name="s")


def _make_sc_agg(with_deg):
  out_type = [jax.ShapeDtypeStruct((_NC * _N_PAD, _D), jnp.float32)]
  if with_deg:
    out_type.append(jax.ShapeDtypeStruct((_NC * _N_PAD,), jnp.float32))

  half = _CPT // 2

  scratch = [
      pltpu.VMEM((half, _CHUNK), jnp.int32),
      pltpu.VMEM((half, _CHUNK), jnp.int32),
      pltpu.VMEM((_CHUNK, _D), jnp.float32),
      pltpu.VMEM((_CHUNK, _D), jnp.float32),
      pltpu.VMEM((_CHUNK,), jnp.float32),
      pltpu.VMEM((_CHUNK,), jnp.float32),
      pltpu.VMEM_SHARED((_N_PAD, _D), jnp.float32),
      pltpu.VMEM_SHARED((_N_PAD,), jnp.float32),
      pltpu.SemaphoreType.DMA,
      pltpu.SemaphoreType.DMA,
      pltpu.SemaphoreType.DMA,
      pltpu.SemaphoreType.DMA,
      pltpu.SemaphoreType.DMA,
  ]

  def body(x_hbm, srcr_hbm, dstr_hbm, *rest):
    if with_deg:
      agg_hbm, deg_hbm = rest[0], rest[1]
      rest = rest[2:]
    else:
      agg_hbm = rest[0]
      rest = rest[1:]
    (idx_s, idx_d, rows0, rows1, vec1d, z1d, agg_sh, deg_sh,
     semg0, semg1, sems0, sems1, semd) = rest
    rows = (rows0, rows1)
    semg = (semg0, semg1)
    sems = (sems0, sems1)

    cid = lax.axis_index("c")
    sid = lax.axis_index("s")
    wid = cid * _NS + sid
    zero16 = jnp.zeros((_LANES,), jnp.float32)
    one16 = jnp.ones((_LANES,), jnp.float32)

    def zrow(r, _):
      for c in range(_D // _LANES):
        rows0[r, pl.ds(c * _LANES, _LANES)] = zero16
      return _
    lax.fori_loop(0, _CHUNK, zrow, 0)

    if with_deg:
      def fill1d(r, _):
        vec1d[pl.ds(r * _LANES, _LANES)] = one16
        z1d[pl.ds(r * _LANES, _LANES)] = zero16
        return _
      lax.fori_loop(0, _CHUNK // _LANES, fill1d, 0)

    row0 = sid * _NROW
    for k in range(_NROW // _CHUNK):
      pltpu.sync_copy(rows0, agg_sh.at[pl.ds(row0 + k * _CHUNK, _CHUNK)])
      if with_deg:
        pltpu.sync_copy(z1d, deg_sh.at[pl.ds(row0 + k * _CHUNK, _CHUNK)])

    plsc.subcore_barrier()

    for h in range(2):
      base = wid * _CPT + h * half
      pltpu.sync_copy(srcr_hbm.at[pl.ds(base, half)], idx_s)
      pltpu.sync_copy(dstr_hbm.at[pl.ds(base, half)], idx_d)

      if with_deg:
        def dstep(j, _):
          pltpu.async_copy(vec1d, deg_sh.at[idx_d.at[j]], semd, add=True)
          return _
        lax.fori_loop(0, half, dstep, 0)

      pltpu.async_copy(x_hbm.at[idx_s.at[0]], rows0, semg0)

      def pair(p, _):
        j0 = 2 * p
        j1 = j0 + 1
        @pl.when(p > 0)
        def _wait_s1():
          pltpu.make_async_copy(rows1, agg_sh.at[idx_d.at[j1 - 2]],
                                sems1).wait()
        pltpu.async_copy(x_hbm.at[idx_s.at[j1]], rows1, semg1)
        pltpu.make_async_copy(x_hbm.at[idx_s.at[j0]], rows0, semg0).wait()
        pltpu.async_copy(rows0, agg_sh.at[idx_d.at[j0]], sems0, add=True)
        pltpu.make_async_copy(x_hbm.at[idx_s.at[j1]], rows1, semg1).wait()
        pltpu.make_async_copy(rows0, agg_sh.at[idx_d.at[j0]], sems0).wait()
        @pl.when(p + 1 < half // 2)
        def _next():
          pltpu.async_copy(x_hbm.at[idx_s.at[j0 + 2]], rows0, semg0)
        pltpu.async_copy(rows1, agg_sh.at[idx_d.at[j1]], sems1, add=True)
        return _
      lax.fori_loop(0, half // 2, pair, 0)

      pltpu.make_async_copy(rows1, agg_sh.at[idx_d.at[half - 1]],
                            sems1).wait()
      if with_deg:
        def ddrain(j, _):
          pltpu.make_async_copy(vec1d, deg_sh.at[idx_d.at[j]], semd).wait()
          return _
        lax.fori_loop(0, half, ddrain, 0)

    plsc.subcore_barrier()

    off = cid * _N_PAD + row0
    pltpu.sync_copy(agg_sh.at[pl.ds(row0, _NROW)], agg_hbm.at[pl.ds(off, _NROW)])
    if with_deg:
      pltpu.sync_copy(deg_sh.at[pl.ds(row0, _NROW)],
                      deg_hbm.at[pl.ds(off, _NROW)])

  return pl.kernel(body, out_type=out_type, mesh=_sc_mesh,
                   scratch_types=scratch)


_sc_agg_deg = _make_sc_agg(True)
_sc_agg = _make_sc_agg(False)

_BLK = 512
_GRID = _N_PAD // _BLK


def _dot(a, w):
  return lax.dot_general(a, w, (((1,), (0,)), ((), ())),
                         precision=lax.Precision.HIGHEST,
                         preferred_element_type=jnp.float32)


def _tc_mid_body(ap, dp, w, bb, o):
  a = ap[0] + ap[1]
  dg = jnp.clip(dp[0] + dp[1], 1.0, None)
  s = a / dg
  o[...] = jnp.tanh(_dot(s, w[...]) + bb[...])


def _tc_final_body(ap, dp, w, bb, xb, gb, betab, o):
  a = ap[0] + ap[1]
  dg = jnp.clip(dp[0] + dp[1], 1.0, None)
  s = a / dg
  h = jnp.tanh(_dot(s, w[...]) + bb[...])
  r = xb[...] + h
  m = jnp.mean(r, axis=1, keepdims=True)
  c = r - m
  v = jnp.mean(c * c, axis=1, keepdims=True)
  o[...] = c * lax.rsqrt(v + 1e-5) * gb[...] + betab[...]


_spec_agg = pl.BlockSpec((2, _BLK, _D), lambda i: (0, i, 0))
_spec_deg = pl.BlockSpec((2, _BLK, 1), lambda i: (0, i, 0))
_spec_w = pl.BlockSpec((_D, _D), lambda i: (0, 0))
_spec_row = pl.BlockSpec((1, _D), lambda i: (0, 0))
_spec_x = pl.BlockSpec((_BLK, _D), lambda i: (i, 0))

_tc_mid = pl.pallas_call(
    _tc_mid_body,
    grid=(_GRID,),
    in_specs=[_spec_agg, _spec_deg, _spec_w, _spec_row],
    out_specs=_spec_x,
    out_shape=jax.ShapeDtypeStruct((_N_PAD, _D), jnp.float32),
)

_tc_final = pl.pallas_call(
    _tc_final_body,
    grid=(_GRID,),
    in_specs=[_spec_agg, _spec_deg, _spec_w, _spec_row, _spec_x, _spec_row,
              _spec_row],
    out_specs=_spec_x,
    out_shape=jax.ShapeDtypeStruct((_N_PAD, _D), jnp.float32),
)


def kernel(x, edge_index, W1, b1, W2, b2, gamma, beta):
  src = edge_index[0]
  dst = edge_index[1]
  pad = _E_PAD - _E
  src_p = jnp.concatenate(
      [src, jnp.zeros((pad,), jnp.int32)]).reshape(_E_PAD // _CHUNK, _CHUNK)
  dump = _N + jnp.arange(pad, dtype=jnp.int32) % (_N_PAD - _N)
  dst_p = jnp.concatenate([dst, dump]).reshape(_E_PAD // _CHUNK, _CHUNK)

  agg1, deg = _sc_agg_deg(x, src_p, dst_p)
  agg1 = agg1.reshape(_NC, _N_PAD, _D)
  deg = deg.reshape(_NC, _N_PAD, 1)

  h1 = _tc_mid(agg1, deg, W1, b1.reshape(1, _D))

  agg2, = _sc_agg(h1, src_p, dst_p)
  agg2 = agg2.reshape(_NC, _N_PAD, _D)

  x_pad = jnp.concatenate([x, jnp.zeros((_N_PAD - _N, _D), jnp.float32)])
  out = _tc_final(agg2, deg, W2, b2.reshape(1, _D), x_pad,
                  gamma.reshape(1, _D), beta.reshape(1, _D))
  return out[:_N]


# speedup vs baseline: 4.3390x; 1.2056x over previous
import functools

import jax
import jax.numpy as jnp
from jax import lax
from jax.experimental import pallas as pl
from jax.experimental.pallas import tpu as pltpu
from jax.experimental.pallas import tpu_sc as plsc

_N = 10000
_E = 320000
_D = 128

_NC = 2
_NS = 16
_NW = _NC * _NS
_CHUNK = 128
_CPT = 80
_E_PAD = _NW * _CPT * _CHUNK
_NROW = 640
_N_PAD = _NS * _NROW

_LANES = 16

_sc_mesh = plsc.VectorSubcoreMesh(core_axis_name="c", subcore_axis_name="s")


def _make_sc_agg(with_deg):
  out_type = [jax.ShapeDtypeStruct((_NC * _N_PAD, _D), jnp.float32)]
  if with_deg:
    out_type.append(jax.ShapeDtypeStruct((_NC * _N_PAD,), jnp.float32))

  half = _CPT // 2

  scratch = [
      pltpu.VMEM((half, _CHUNK), jnp.int32),
      pltpu.VMEM((half, _CHUNK), jnp.int32),
      pltpu.VMEM((_CHUNK, _D), jnp.float32),
      pltpu.VMEM((_CHUNK, _D), jnp.float32),
      pltpu.VMEM((_CHUNK,), jnp.float32),
      pltpu.VMEM((_CHUNK,), jnp.float32),
      pltpu.VMEM_SHARED((_N_PAD, _D), jnp.float32),
      pltpu.VMEM_SHARED((_N_PAD,), jnp.float32),
      pltpu.SemaphoreType.DMA,
      pltpu.SemaphoreType.DMA,
      pltpu.SemaphoreType.DMA,
      pltpu.SemaphoreType.DMA,
      pltpu.SemaphoreType.DMA,
  ]

  def body(x_hbm, srcr_hbm, dstr_hbm, *rest):
    if with_deg:
      agg_hbm, deg_hbm = rest[0], rest[1]
      rest = rest[2:]
    else:
      agg_hbm = rest[0]
      rest = rest[1:]
    (idx_s, idx_d, rows0, rows1, vec1d, z1d, agg_sh, deg_sh,
     semg0, semg1, sems0, sems1, semd) = rest
    rows = (rows0, rows1)
    semg = (semg0, semg1)
    sems = (sems0, sems1)

    cid = lax.axis_index("c")
    sid = lax.axis_index("s")
    wid = cid * _NS + sid
    zero16 = jnp.zeros((_LANES,), jnp.float32)
    one16 = jnp.ones((_LANES,), jnp.float32)

    def zrow(r, _):
      for c in range(_D // _LANES):
        rows0[r, pl.ds(c * _LANES, _LANES)] = zero16
      return _
    lax.fori_loop(0, _CHUNK, zrow, 0)

    if with_deg:
      def fill1d(r, _):
        vec1d[pl.ds(r * _LANES, _LANES)] = one16
        z1d[pl.ds(r * _LANES, _LANES)] = zero16
        return _
      lax.fori_loop(0, _CHUNK // _LANES, fill1d, 0)

    row0 = sid * _NROW
    for k in range(_NROW // _CHUNK):
      pltpu.sync_copy(rows0, agg_sh.at[pl.ds(row0 + k * _CHUNK, _CHUNK)])
      if with_deg:
        pltpu.sync_copy(z1d, deg_sh.at[pl.ds(row0 + k * _CHUNK, _CHUNK)])

    plsc.subcore_barrier()

    for h in range(2):
      base = wid * _CPT + h * half
      pltpu.sync_copy(srcr_hbm.at[pl.ds(base, half)], idx_s)
      pltpu.sync_copy(dstr_hbm.at[pl.ds(base, half)], idx_d)

      if with_deg:
        def dstep(j, _):
          pltpu.async_copy(vec1d, deg_sh.at[idx_d.at[j]], semd, add=True)
          return _
        lax.fori_loop(0, half, dstep, 0)

      pltpu.async_copy(x_hbm.at[idx_s.at[0]], rows0, semg0)

      def pair(p, _):
        j0 = 2 * p
        j1 = j0 + 1
        @pl.when(p > 0)
        def _wait_s1():
          pltpu.make_async_copy(rows1, agg_sh.at[idx_d.at[j1 - 2]],
                                sems1).wait()
        pltpu.async_copy(x_hbm.at[idx_s.at[j1]], rows1, semg1)
        pltpu.make_async_copy(x_hbm.at[idx_s.at[j0]], rows0, semg0).wait()
        pltpu.async_copy(rows0, agg_sh.at[idx_d.at[j0]], sems0, add=True)
        pltpu.make_async_copy(x_hbm.at[idx_s.at[j1]], rows1, semg1).wait()
        pltpu.make_async_copy(rows0, agg_sh.at[idx_d.at[j0]], sems0).wait()
        @pl.when(p + 1 < half // 2)
        def _next():
          pltpu.async_copy(x_hbm.at[idx_s.at[j0 + 2]], rows0, semg0)
        pltpu.async_copy(rows1, agg_sh.at[idx_d.at[j1]], sems1, add=True)
        return _
      lax.fori_loop(0, half // 2, pair, 0)

      pltpu.make_async_copy(rows1, agg_sh.at[idx_d.at[half - 1]],
                            sems1).wait()
      if with_deg:
        def ddrain(j, _):
          pltpu.make_async_copy(vec1d, deg_sh.at[idx_d.at[j]], semd).wait()
          return _
        lax.fori_loop(0, half, ddrain, 0)

    plsc.subcore_barrier()

    off = cid * _N_PAD + row0
    pltpu.sync_copy(agg_sh.at[pl.ds(row0, _NROW)], agg_hbm.at[pl.ds(off, _NROW)])
    if with_deg:
      pltpu.sync_copy(deg_sh.at[pl.ds(row0, _NROW)],
                      deg_hbm.at[pl.ds(off, _NROW)])

  return pl.kernel(body, out_type=out_type, mesh=_sc_mesh,
                   scratch_types=scratch)


_sc_agg_deg = _make_sc_agg(True)
_sc_agg = _make_sc_agg(False)

_BLK = 512
_GRID = _N_PAD // _BLK


def _dot(a, w):
  return lax.dot_general(a, w, (((1,), (0,)), ((), ())),
                         precision=lax.Precision.HIGHEST,
                         preferred_element_type=jnp.float32)


def _tc_mid_body(ap, dp, w, bb, o):
  a = ap[0] + ap[1]
  dg = jnp.clip(dp[0] + dp[1], 1.0, None)
  s = a / dg
  o[...] = jnp.tanh(_dot(s, w[...]) + bb[...])


def _tc_final_body(ap, dp, w, bb, xb, gb, betab, o):
  a = ap[0] + ap[1]
  dg = jnp.clip(dp[0] + dp[1], 1.0, None)
  s = a / dg
  h = jnp.tanh(_dot(s, w[...]) + bb[...])
  r = xb[...] + h
  m = jnp.mean(r, axis=1, keepdims=True)
  c = r - m
  v = jnp.mean(c * c, axis=1, keepdims=True)
  o[...] = c * lax.rsqrt(v + 1e-5) * gb[...] + betab[...]


_spec_agg = pl.BlockSpec((2, _BLK, _D), lambda i: (0, i, 0))
_spec_deg = pl.BlockSpec((2, _BLK, 1), lambda i: (0, i, 0))
_spec_w = pl.BlockSpec((_D, _D), lambda i: (0, 0))
_spec_row = pl.BlockSpec((1, _D), lambda i: (0, 0))
_spec_x = pl.BlockSpec((_BLK, _D), lambda i: (i, 0))

_tc_mid = pl.pallas_call(
    _tc_mid_body,
    grid=(_GRID,),
    in_specs=[_spec_agg, _spec_deg, _spec_w, _spec_row],
    out_specs=_spec_x,
    out_shape=jax.ShapeDtypeStruct((_N_PAD, _D), jnp.float32),
)

_tc_final = pl.pallas_call(
    _tc_final_body,
    grid=(_GRID,),
    in_specs=[_spec_agg, _spec_deg, _spec_w, _spec_row, _spec_x, _spec_row,
              _spec_row],
    out_specs=_spec_x,
    out_shape=jax.ShapeDtypeStruct((_N_PAD, _D), jnp.float32),
)


def kernel(x, edge_index, W1, b1, W2, b2, gamma, beta):
  src = edge_index[0]
  dst = edge_index[1]
  ppt = _E_PAD // _NW - _E // _NW
  src_p = jnp.concatenate(
      [src.reshape(_NW, _E // _NW),
       jnp.zeros((_NW, ppt), jnp.int32)], axis=1)
  src_p = src_p.reshape(_E_PAD // _CHUNK, _CHUNK)
  dump = _N + jnp.arange(_NW * ppt, dtype=jnp.int32) % (_N_PAD - _N)
  dst_p = jnp.concatenate(
      [dst.reshape(_NW, _E // _NW), dump.reshape(_NW, ppt)], axis=1)
  dst_p = dst_p.reshape(_E_PAD // _CHUNK, _CHUNK)

  agg1, deg = _sc_agg_deg(x, src_p, dst_p)
  agg1 = agg1.reshape(_NC, _N_PAD, _D)
  deg = deg.reshape(_NC, _N_PAD, 1)

  h1 = _tc_mid(agg1, deg, W1, b1.reshape(1, _D))

  agg2, = _sc_agg(h1, src_p, dst_p)
  agg2 = agg2.reshape(_NC, _N_PAD, _D)

  x_pad = jnp.concatenate([x, jnp.zeros((_N_PAD - _N, _D), jnp.float32)])
  out = _tc_final(agg2, deg, W2, b2.reshape(1, _D), x_pad,
                  gamma.reshape(1, _D), beta.reshape(1, _D))
  return out[:_N]
